# baseline (speedup 1.0000x reference)
import functools

import jax
import jax.numpy as jnp
from jax import lax
from jax.experimental import pallas as pl
from jax.experimental.pallas import tpu as pltpu
from jax.experimental.pallas import tpu_sc as plsc

N_BLOCK = 2000
NEG_SLOPE = 0.01

N_NODES = 100000
TILE_ROWS = 6500
N_ACC = 16 * TILE_ROWS
E_GROUP = 128
CHUNK_G = 4


def _lrelu(x):
    return jnp.where(x > 0, x, NEG_SLOPE * x)


def _seg16_body(q_hbm, src_hbm, dst_hbm, zeros_hbm, out_hbm,
                sidx0, didx0, rows0, sidx1, didx1, rows1, acc,
                gsem0, ssem0, isem0, gsem1, ssem1, isem1):
    c = lax.axis_index("c")
    s = lax.axis_index("s")
    n_groups = src_hbm.shape[0]
    gpt = n_groups // 16
    n_chunks = gpt // CHUNK_G
    n_pairs = n_chunks // 2

    pltpu.sync_copy(zeros_hbm.at[pl.ds(0, TILE_ROWS)],
                    acc.at[pl.ds(s * TILE_ROWS, TILE_ROWS)])
    plsc.subcore_barrier()

    row_off = c * N_NODES
    bufs = ((sidx0, didx0, rows0, gsem0, ssem0, isem0),
            (sidx1, didx1, rows1, gsem1, ssem1, isem1))

    def fire_idx(cj, b):
        sidx, didx, _, _, _, isem = bufs[b]
        gb = s * gpt + cj * CHUNK_G
        pltpu.async_copy(src_hbm.at[pl.ds(gb, CHUNK_G)], sidx, isem)
        pltpu.async_copy(dst_hbm.at[pl.ds(gb, CHUNK_G)], didx, isem)

    def finish_and_fire(b):
        sidx, didx, rows, gsem, _, isem = bufs[b]
        pltpu.make_async_copy(src_hbm.at[pl.ds(0, CHUNK_G)], sidx,
                              isem).wait()
        pltpu.make_async_copy(src_hbm.at[pl.ds(0, CHUNK_G)], didx,
                              isem).wait()
        for g in range(CHUNK_G):
            for k in range(E_GROUP // 16):
                v = sidx[g, pl.ds(k * 16, 16)]
                sidx[g, pl.ds(k * 16, 16)] = v + row_off
        return [pltpu.async_copy(q_hbm.at[sidx.at[g]],
                                 rows.at[pl.ds(g * E_GROUP, E_GROUP)], gsem)
                for g in range(CHUNK_G)]

    def fire_scatters(b):
        _, didx, rows, _, ssem, _ = bufs[b]
        return [pltpu.async_copy(rows.at[pl.ds(g * E_GROUP, E_GROUP)],
                                 acc.at[didx.at[g]], ssem, add=True)
                for g in range(CHUNK_G)]

    fire_idx(0, 0)
    g0 = finish_and_fire(0)
    fire_idx(1, 1)
    for cp in g0:
        cp.wait()

    def pair(j0, carry):
        s0 = fire_scatters(0)
        g1 = finish_and_fire(1)
        for cp in s0:
            cp.wait()

        @pl.when(j0 + 1 < n_pairs)
        def _():
            fire_idx(2 * j0 + 2, 0)
            for cp in g1:
                cp.wait()
            s1 = fire_scatters(1)
            gn = finish_and_fire(0)
            for cp in s1:
                cp.wait()
            fire_idx(2 * j0 + 3, 1)
            for cp in gn:
                cp.wait()

        @pl.when(j0 + 1 >= n_pairs)
        def _():
            for cp in g1:
                cp.wait()
            s1 = fire_scatters(1)
            for cp in s1:
                cp.wait()

        return carry

    lax.fori_loop(0, n_pairs, pair, 0)
    plsc.subcore_barrier()
    pltpu.sync_copy(acc.at[pl.ds(s * TILE_ROWS, TILE_ROWS)],
                    out_hbm.at[pl.ds(c * N_ACC + s * TILE_ROWS, TILE_ROWS)])


def _sc_seg16(q_flat, src2, dst2, zeros16):
    mesh = plsc.VectorSubcoreMesh(core_axis_name="c", subcore_axis_name="s")
    fn = pl.kernel(
        _seg16_body,
        out_type=jax.ShapeDtypeStruct((2 * N_ACC, 16), jnp.float32),
        mesh=mesh,
        compiler_params=pltpu.CompilerParams(use_tc_tiling_on_sc=False),
        scratch_types=[
            pltpu.VMEM((CHUNK_G, E_GROUP), jnp.int32),
            pltpu.VMEM((CHUNK_G, E_GROUP), jnp.int32),
            pltpu.VMEM((CHUNK_G * E_GROUP, 16), jnp.float32),
            pltpu.VMEM((CHUNK_G, E_GROUP), jnp.int32),
            pltpu.VMEM((CHUNK_G, E_GROUP), jnp.int32),
            pltpu.VMEM((CHUNK_G * E_GROUP, 16), jnp.float32),
            pltpu.VMEM_SHARED((N_ACC, 16), jnp.float32),
            pltpu.SemaphoreType.DMA,
            pltpu.SemaphoreType.DMA,
            pltpu.SemaphoreType.DMA,
            pltpu.SemaphoreType.DMA,
            pltpu.SemaphoreType.DMA,
            pltpu.SemaphoreType.DMA,
        ],
    )
    return fn(q_flat, src2, dst2, zeros16)


def _deg_body(dst_hbm, ones_hbm, zeros_hbm, out_hbm,
              didx, ones_v, acc, ssem):
    c = lax.axis_index("c")
    s = lax.axis_index("s")
    w = c * 16 + s
    n_groups = dst_hbm.shape[0]
    gpt = n_groups // 32
    n_chunks = gpt // CHUNK_G

    pltpu.sync_copy(zeros_hbm.at[pl.ds(0, TILE_ROWS)],
                    acc.at[pl.ds(s * TILE_ROWS, TILE_ROWS)])
    pltpu.sync_copy(ones_hbm, ones_v)
    plsc.subcore_barrier()

    def chunk(j, carry):
        gb = w * gpt + j * CHUNK_G
        pltpu.sync_copy(dst_hbm.at[pl.ds(gb, CHUNK_G)], didx)
        scps = [pltpu.async_copy(ones_v, acc.at[didx.at[g]], ssem, add=True)
                for g in range(CHUNK_G)]
        for cp in scps:
            cp.wait()
        return carry

    lax.fori_loop(0, n_chunks, chunk, 0)
    plsc.subcore_barrier()
    pltpu.sync_copy(acc.at[pl.ds(s * TILE_ROWS, TILE_ROWS)],
                    out_hbm.at[pl.ds(c * N_ACC + s * TILE_ROWS, TILE_ROWS)])


def _sc_deg(dst2, ones8, zeros8):
    mesh = plsc.VectorSubcoreMesh(core_axis_name="c", subcore_axis_name="s")
    fn = pl.kernel(
        _deg_body,
        out_type=jax.ShapeDtypeStruct((2 * N_ACC, 8), jnp.float32),
        mesh=mesh,
        compiler_params=pltpu.CompilerParams(use_tc_tiling_on_sc=False),
        scratch_types=[
            pltpu.VMEM((CHUNK_G, E_GROUP), jnp.int32),
            pltpu.VMEM((E_GROUP, 8), jnp.float32),
            pltpu.VMEM_SHARED((N_ACC, 8), jnp.float32),
            pltpu.SemaphoreType.DMA,
        ],
    )
    return fn(dst2, ones8, zeros8)


def _front_body(real_ref, cat_ref, w1a_ref, t1_ref, q1_ref):
    real = real_ref[...]
    cat = cat_ref[...]
    n_cat = t1_ref.shape[1]
    onehot = (cat == lax.broadcasted_iota(jnp.int32, (1, n_cat), 1)
              ).astype(jnp.float32)
    q1 = jnp.dot(real, w1a_ref[0], preferred_element_type=jnp.float32)
    q1 += jnp.dot(onehot, t1_ref[0], preferred_element_type=jnp.float32)
    q1_ref[...] = q1


def _front(real, cat, w1a_s, t1_s, blk):
    n = real.shape[0]
    nb = n // blk
    return pl.pallas_call(
        _front_body,
        grid=(2, nb),
        in_specs=[
            pl.BlockSpec((blk, real.shape[1]), lambda h, i: (i, 0)),
            pl.BlockSpec((blk, 1), lambda h, i: (i, 0)),
            pl.BlockSpec((1,) + w1a_s.shape[1:], lambda h, i: (h, 0, 0)),
            pl.BlockSpec((1,) + t1_s.shape[1:], lambda h, i: (h, 0, 0)),
        ],
        out_specs=pl.BlockSpec((blk, 16), lambda h, i: (h * nb + i, 0)),
        out_shape=jax.ShapeDtypeStruct((2 * n, 16), jnp.float32),
    )(real, cat, w1a_s, t1_s)


def _mid_body(lo_ref, hi_ref, d0_ref, d1_ref, b1_ref, w2_ref, q2_ref):
    agg = jnp.concatenate([lo_ref[...], hi_ref[...]], axis=1)
    deg = jnp.maximum(d0_ref[...][:, :1] + d1_ref[...][:, :1], 1.0)
    h1 = _lrelu(agg / deg + b1_ref[...])
    q2_ref[...] = jnp.dot(h1, w2_ref[0], preferred_element_type=jnp.float32)


def _mid(agg1, degp, b1, w2_s, blk):
    n = N_NODES
    nb = n // blk
    off = N_ACC // blk
    return pl.pallas_call(
        _mid_body,
        grid=(2, nb),
        in_specs=[
            pl.BlockSpec((blk, 16), lambda h, i: (i, 0)),
            pl.BlockSpec((blk, 16), lambda h, i: (i + off, 0)),
            pl.BlockSpec((blk, 8), lambda h, i: (i, 0)),
            pl.BlockSpec((blk, 8), lambda h, i: (i + off, 0)),
            pl.BlockSpec((1, b1.shape[1]), lambda h, i: (0, 0)),
            pl.BlockSpec((1,) + w2_s.shape[1:], lambda h, i: (h, 0, 0)),
        ],
        out_specs=pl.BlockSpec((blk, 16), lambda h, i: (h * nb + i, 0)),
        out_shape=jax.ShapeDtypeStruct((2 * n, 16), jnp.float32),
    )(agg1, agg1, degp, degp, b1, w2_s)


def _head_body(lo_ref, hi_ref, d0_ref, d1_ref, mask_ref, b2_ref,
               f1w_ref, f1b_ref, f2w_ref, f2b_ref, f3w_ref, f3b_ref,
               logits_ref, m_ref, s_ref, m_sc, s_sc):
    i = pl.program_id(0)

    @pl.when(i == 0)
    def _():
        m_sc[0] = -jnp.inf
        s_sc[0] = 0.0

    agg = jnp.concatenate([lo_ref[...], hi_ref[...]], axis=1)
    deg = jnp.maximum(d0_ref[...][:, :1] + d1_ref[...][:, :1], 1.0)
    embs = agg / deg + b2_ref[...]
    x = _lrelu(jnp.dot(embs, f1w_ref[...], preferred_element_type=jnp.float32)
               + f1b_ref[...])
    x = _lrelu(jnp.dot(x, f2w_ref[...], preferred_element_type=jnp.float32)
               + f2b_ref[...])
    logit = jnp.dot(x, f3w_ref[...], preferred_element_type=jnp.float32) \
        + f3b_ref[...]
    logit = jnp.where(mask_ref[...], logit, -jnp.inf)
    logits_ref[...] = logit

    blk_max = jnp.max(logit)
    m_old = m_sc[0]
    m_new = jnp.maximum(m_old, blk_max)
    scale = jnp.where(m_new == -jnp.inf, 0.0, jnp.exp(m_old - m_new))
    blk_sum = jnp.sum(jnp.where(logit == -jnp.inf, 0.0,
                                jnp.exp(logit - m_new)))
    s_sc[0] = s_sc[0] * scale + blk_sum
    m_sc[0] = m_new

    @pl.when(i == pl.num_programs(0) - 1)
    def _():
        m_ref[0, 0] = m_sc[0]
        s_ref[0, 0] = s_sc[0]


def _head(agg2, degp, mask, b2, f1w, f1b, f2w, f2b, f3w, f3b, blk):
    n = N_NODES
    off = N_ACC // blk
    return pl.pallas_call(
        _head_body,
        grid=(n // blk,),
        in_specs=[
            pl.BlockSpec((blk, 16), lambda i: (i, 0)),
            pl.BlockSpec((blk, 16), lambda i: (i + off, 0)),
            pl.BlockSpec((blk, 8), lambda i: (i, 0)),
            pl.BlockSpec((blk, 8), lambda i: (i + off, 0)),
            pl.BlockSpec((blk, 1), lambda i: (i, 0)),
            pl.BlockSpec((1, b2.shape[1]), lambda i: (0, 0)),
            pl.BlockSpec(f1w.shape, lambda i: (0, 0)),
            pl.BlockSpec((1, f1b.shape[1]), lambda i: (0, 0)),
            pl.BlockSpec(f2w.shape, lambda i: (0, 0)),
            pl.BlockSpec((1, f2b.shape[1]), lambda i: (0, 0)),
            pl.BlockSpec(f3w.shape, lambda i: (0, 0)),
            pl.BlockSpec((1, 1), lambda i: (0, 0)),
        ],
        out_specs=[
            pl.BlockSpec((blk, 1), lambda i: (i, 0)),
            pl.BlockSpec(memory_space=pltpu.SMEM),
            pl.BlockSpec(memory_space=pltpu.SMEM),
        ],
        out_shape=[
            jax.ShapeDtypeStruct((n, 1), jnp.float32),
            jax.ShapeDtypeStruct((1, 1), jnp.float32),
            jax.ShapeDtypeStruct((1, 1), jnp.float32),
        ],
        scratch_shapes=[pltpu.SMEM((1,), jnp.float32),
                        pltpu.SMEM((1,), jnp.float32)],
    )(agg2, agg2, degp, degp, mask, b2, f1w, f1b, f2w, f2b, f3w, f3b)


def _norm_body(logits_ref, m_ref, s_ref, probs_ref):
    logit = logits_ref[...]
    m = m_ref[0, 0]
    s = s_ref[0, 0]
    e = jnp.where(logit == -jnp.inf, 0.0, jnp.exp(logit - m))
    probs_ref[...] = e / s


def _norm(logits, m, s, blk):
    n = logits.shape[0]
    return pl.pallas_call(
        _norm_body,
        grid=(n // blk,),
        in_specs=[
            pl.BlockSpec((blk, 1), lambda i: (i, 0)),
            pl.BlockSpec(memory_space=pltpu.SMEM),
            pl.BlockSpec(memory_space=pltpu.SMEM),
        ],
        out_specs=pl.BlockSpec((blk, 1), lambda i: (i, 0)),
        out_shape=jax.ShapeDtypeStruct((n, 1), jnp.float32),
    )(logits, m, s)


def kernel(real_features, cat_features, edge_index, mask, emb_table,
           W1, b1, W2, b2, fc1_w, fc1_b, fc2_w, fc2_b, fc3_w, fc3_b):
    n = real_features.shape[0]
    r = real_features.shape[1]
    e = edge_index.shape[1]
    blk = N_BLOCK if n % N_BLOCK == 0 else n
    cat = cat_features.astype(jnp.int32)
    src = edge_index[0].astype(jnp.int32)
    dst = edge_index[1].astype(jnp.int32)

    grp = 32 * 2 * CHUNK_G * E_GROUP
    e_pad = ((e + grp - 1) // grp) * grp
    src_p = jnp.concatenate([src, jnp.zeros((e_pad - e,), jnp.int32)])
    dst_p = jnp.concatenate([dst, jnp.full((e_pad - e,), n, jnp.int32)])
    src2 = src_p.reshape(e_pad // E_GROUP, E_GROUP)
    dst2 = dst_p.reshape(e_pad // E_GROUP, E_GROUP)

    zeros16 = jnp.zeros((TILE_ROWS, 16), jnp.float32)
    zeros8 = jnp.zeros((TILE_ROWS, 8), jnp.float32)
    ones8 = jnp.ones((E_GROUP, 8), jnp.float32)

    t1 = emb_table @ W1[r:]
    w1a = W1[:r]
    w1a_s = jnp.stack([w1a[:, :16], w1a[:, 16:]])
    t1_s = jnp.stack([t1[:, :16], t1[:, 16:]])
    w2_s = jnp.stack([W2[:, :16], W2[:, 16:]])

    degp = _sc_deg(dst2, ones8, zeros8)

    q1_flat = _front(real_features, cat, w1a_s, t1_s, blk)
    agg1 = _sc_seg16(q1_flat, src2, dst2, zeros16)

    q2_flat = _mid(agg1, degp, b1[None, :], w2_s, blk)
    agg2 = _sc_seg16(q2_flat, src2, dst2, zeros16)

    logits, m, s = _head(agg2, degp, mask, b2[None, :],
                         fc1_w, fc1_b[None, :], fc2_w, fc2_b[None, :],
                         fc3_w, fc3_b[None, :], blk)
    probs = _norm(logits, m, s, blk)
    return probs[:, 0]


# speedup vs baseline: 1.0179x; 1.0179x over previous
import functools

import jax
import jax.numpy as jnp
from jax import lax
from jax.experimental import pallas as pl
from jax.experimental.pallas import tpu as pltpu
from jax.experimental.pallas import tpu_sc as plsc

N_BLOCK = 2000
NEG_SLOPE = 0.01

N_NODES = 100000
TILE_ROWS = 6500
N_ACC = 16 * TILE_ROWS
E_GROUP = 128
CHUNK_G = 4


def _lrelu(x):
    return jnp.where(x > 0, x, NEG_SLOPE * x)


def _seg16_body(q_hbm, src_hbm, dst_hbm, zeros_hbm, out_hbm,
                sidx0, didx0, rows0, sidx1, didx1, rows1, acc,
                gsem0, ssem0, isem0, gsem1, ssem1, isem1):
    c = lax.axis_index("c")
    s = lax.axis_index("s")
    n_groups = src_hbm.shape[0]
    gpt = n_groups // 16
    n_chunks = gpt // CHUNK_G
    n_pairs = n_chunks // 2

    pltpu.sync_copy(zeros_hbm.at[pl.ds(0, TILE_ROWS)],
                    acc.at[pl.ds(s * TILE_ROWS, TILE_ROWS)])
    plsc.subcore_barrier()

    row_off = c * N_NODES
    bufs = ((sidx0, didx0, rows0, gsem0, ssem0, isem0),
            (sidx1, didx1, rows1, gsem1, ssem1, isem1))

    def load_and_fire(cj, sidx, didx, rows, gsem, isem):
        gb = s * gpt + cj * CHUNK_G
        ic0 = pltpu.async_copy(src_hbm.at[pl.ds(gb, CHUNK_G)], sidx, isem)
        ic1 = pltpu.async_copy(dst_hbm.at[pl.ds(gb, CHUNK_G)], didx, isem)
        ic0.wait()
        ic1.wait()
        for g in range(CHUNK_G):
            for k in range(E_GROUP // 16):
                v = sidx[g, pl.ds(k * 16, 16)]
                sidx[g, pl.ds(k * 16, 16)] = v + row_off
        return [pltpu.async_copy(q_hbm.at[sidx.at[g]],
                                 rows.at[pl.ds(g * E_GROUP, E_GROUP)], gsem)
                for g in range(CHUNK_G)]

    def fire_scatters(didx, rows, ssem):
        return [pltpu.async_copy(rows.at[pl.ds(g * E_GROUP, E_GROUP)],
                                 acc.at[didx.at[g]], ssem, add=True)
                for g in range(CHUNK_G)]

    g0 = load_and_fire(0, bufs[0][0], bufs[0][1], bufs[0][2], bufs[0][3],
                       bufs[0][5])
    for cp in g0:
        cp.wait()

    def pair(j0, carry):
        s0 = fire_scatters(bufs[0][1], bufs[0][2], bufs[0][4])
        g1 = load_and_fire(2 * j0 + 1, bufs[1][0], bufs[1][1], bufs[1][2],
                           bufs[1][3], bufs[1][5])
        for cp in s0:
            cp.wait()

        @pl.when(j0 + 1 < n_pairs)
        def _():
            gn = load_and_fire(2 * j0 + 2, bufs[0][0], bufs[0][1],
                               bufs[0][2], bufs[0][3], bufs[0][5])
            for cp in g1:
                cp.wait()
            s1 = fire_scatters(bufs[1][1], bufs[1][2], bufs[1][4])
            for cp in s1:
                cp.wait()
            for cp in gn:
                cp.wait()

        @pl.when(j0 + 1 >= n_pairs)
        def _():
            for cp in g1:
                cp.wait()
            s1 = fire_scatters(bufs[1][1], bufs[1][2], bufs[1][4])
            for cp in s1:
                cp.wait()

        return carry

    lax.fori_loop(0, n_pairs, pair, 0)
    plsc.subcore_barrier()
    pltpu.sync_copy(acc.at[pl.ds(s * TILE_ROWS, TILE_ROWS)],
                    out_hbm.at[pl.ds(c * N_ACC + s * TILE_ROWS, TILE_ROWS)])


def _sc_seg16(q_flat, src2, dst2, zeros16):
    mesh = plsc.VectorSubcoreMesh(core_axis_name="c", subcore_axis_name="s")
    fn = pl.kernel(
        _seg16_body,
        out_type=jax.ShapeDtypeStruct((2 * N_ACC, 16), jnp.float32),
        mesh=mesh,
        compiler_params=pltpu.CompilerParams(use_tc_tiling_on_sc=False),
        scratch_types=[
            pltpu.VMEM((CHUNK_G, E_GROUP), jnp.int32),
            pltpu.VMEM((CHUNK_G, E_GROUP), jnp.int32),
            pltpu.VMEM((CHUNK_G * E_GROUP, 16), jnp.float32),
            pltpu.VMEM((CHUNK_G, E_GROUP), jnp.int32),
            pltpu.VMEM((CHUNK_G, E_GROUP), jnp.int32),
            pltpu.VMEM((CHUNK_G * E_GROUP, 16), jnp.float32),
            pltpu.VMEM_SHARED((N_ACC, 16), jnp.float32),
            pltpu.SemaphoreType.DMA,
            pltpu.SemaphoreType.DMA,
            pltpu.SemaphoreType.DMA,
            pltpu.SemaphoreType.DMA,
            pltpu.SemaphoreType.DMA,
            pltpu.SemaphoreType.DMA,
        ],
    )
    return fn(q_flat, src2, dst2, zeros16)


def _deg_body(dst_hbm, ones_hbm, zeros_hbm, out_hbm,
              didx, ones_v, acc, ssem):
    c = lax.axis_index("c")
    s = lax.axis_index("s")
    w = c * 16 + s
    n_groups = dst_hbm.shape[0]
    gpt = n_groups // 32
    n_chunks = gpt // CHUNK_G

    pltpu.sync_copy(zeros_hbm.at[pl.ds(0, TILE_ROWS)],
                    acc.at[pl.ds(s * TILE_ROWS, TILE_ROWS)])
    pltpu.sync_copy(ones_hbm, ones_v)
    plsc.subcore_barrier()

    def chunk(j, carry):
        gb = w * gpt + j * CHUNK_G
        pltpu.sync_copy(dst_hbm.at[pl.ds(gb, CHUNK_G)], didx)
        scps = [pltpu.async_copy(ones_v, acc.at[didx.at[g]], ssem, add=True)
                for g in range(CHUNK_G)]
        for cp in scps:
            cp.wait()
        return carry

    lax.fori_loop(0, n_chunks, chunk, 0)
    plsc.subcore_barrier()
    pltpu.sync_copy(acc.at[pl.ds(s * TILE_ROWS, TILE_ROWS)],
                    out_hbm.at[pl.ds(c * N_ACC + s * TILE_ROWS, TILE_ROWS)])


def _sc_deg(dst2, ones8, zeros8):
    mesh = plsc.VectorSubcoreMesh(core_axis_name="c", subcore_axis_name="s")
    fn = pl.kernel(
        _deg_body,
        out_type=jax.ShapeDtypeStruct((2 * N_ACC, 8), jnp.float32),
        mesh=mesh,
        compiler_params=pltpu.CompilerParams(use_tc_tiling_on_sc=False),
        scratch_types=[
            pltpu.VMEM((CHUNK_G, E_GROUP), jnp.int32),
            pltpu.VMEM((E_GROUP, 8), jnp.float32),
            pltpu.VMEM_SHARED((N_ACC, 8), jnp.float32),
            pltpu.SemaphoreType.DMA,
        ],
    )
    return fn(dst2, ones8, zeros8)


def _front_body(real_ref, cat_ref, w1a_ref, t1_ref, q1_ref):
    real = real_ref[...]
    cat = cat_ref[...]
    n_cat = t1_ref.shape[1]
    onehot = (cat == lax.broadcasted_iota(jnp.int32, (1, n_cat), 1)
              ).astype(jnp.float32)
    q1 = jnp.dot(real, w1a_ref[0], preferred_element_type=jnp.float32)
    q1 += jnp.dot(onehot, t1_ref[0], preferred_element_type=jnp.float32)
    q1_ref[...] = q1


def _front(real, cat, w1a_s, t1_s, blk):
    n = real.shape[0]
    nb = n // blk
    return pl.pallas_call(
        _front_body,
        grid=(2, nb),
        in_specs=[
            pl.BlockSpec((blk, real.shape[1]), lambda h, i: (i, 0)),
            pl.BlockSpec((blk, 1), lambda h, i: (i, 0)),
            pl.BlockSpec((1,) + w1a_s.shape[1:], lambda h, i: (h, 0, 0)),
            pl.BlockSpec((1,) + t1_s.shape[1:], lambda h, i: (h, 0, 0)),
        ],
        out_specs=pl.BlockSpec((blk, 16), lambda h, i: (h * nb + i, 0)),
        out_shape=jax.ShapeDtypeStruct((2 * n, 16), jnp.float32),
    )(real, cat, w1a_s, t1_s)


def _mid_body(lo_ref, hi_ref, d0_ref, d1_ref, b1_ref, w2_ref, q2_ref):
    agg = jnp.concatenate([lo_ref[...], hi_ref[...]], axis=1)
    deg = jnp.maximum(d0_ref[...][:, :1] + d1_ref[...][:, :1], 1.0)
    h1 = _lrelu(agg / deg + b1_ref[...])
    q2_ref[...] = jnp.dot(h1, w2_ref[0], preferred_element_type=jnp.float32)


def _mid(agg1, degp, b1, w2_s, blk):
    n = N_NODES
    nb = n // blk
    off = N_ACC // blk
    return pl.pallas_call(
        _mid_body,
        grid=(2, nb),
        in_specs=[
            pl.BlockSpec((blk, 16), lambda h, i: (i, 0)),
            pl.BlockSpec((blk, 16), lambda h, i: (i + off, 0)),
            pl.BlockSpec((blk, 8), lambda h, i: (i, 0)),
            pl.BlockSpec((blk, 8), lambda h, i: (i + off, 0)),
            pl.BlockSpec((1, b1.shape[1]), lambda h, i: (0, 0)),
            pl.BlockSpec((1,) + w2_s.shape[1:], lambda h, i: (h, 0, 0)),
        ],
        out_specs=pl.BlockSpec((blk, 16), lambda h, i: (h * nb + i, 0)),
        out_shape=jax.ShapeDtypeStruct((2 * n, 16), jnp.float32),
    )(agg1, agg1, degp, degp, b1, w2_s)


def _head_body(lo_ref, hi_ref, d0_ref, d1_ref, mask_ref, b2_ref,
               f1w_ref, f1b_ref, f2w_ref, f2b_ref, f3w_ref, f3b_ref,
               logits_ref, m_ref, s_ref, m_sc, s_sc):
    i = pl.program_id(0)

    @pl.when(i == 0)
    def _():
        m_sc[0] = -jnp.inf
        s_sc[0] = 0.0

    agg = jnp.concatenate([lo_ref[...], hi_ref[...]], axis=1)
    deg = jnp.maximum(d0_ref[...][:, :1] + d1_ref[...][:, :1], 1.0)
    embs = agg / deg + b2_ref[...]
    x = _lrelu(jnp.dot(embs, f1w_ref[...], preferred_element_type=jnp.float32)
               + f1b_ref[...])
    x = _lrelu(jnp.dot(x, f2w_ref[...], preferred_element_type=jnp.float32)
               + f2b_ref[...])
    logit = jnp.dot(x, f3w_ref[...], preferred_element_type=jnp.float32) \
        + f3b_ref[...]
    logit = jnp.where(mask_ref[...], logit, -jnp.inf)
    logits_ref[...] = logit

    blk_max = jnp.max(logit)
    m_old = m_sc[0]
    m_new = jnp.maximum(m_old, blk_max)
    scale = jnp.where(m_new == -jnp.inf, 0.0, jnp.exp(m_old - m_new))
    blk_sum = jnp.sum(jnp.where(logit == -jnp.inf, 0.0,
                                jnp.exp(logit - m_new)))
    s_sc[0] = s_sc[0] * scale + blk_sum
    m_sc[0] = m_new

    @pl.when(i == pl.num_programs(0) - 1)
    def _():
        m_ref[0, 0] = m_sc[0]
        s_ref[0, 0] = s_sc[0]


def _head(agg2, degp, mask, b2, f1w, f1b, f2w, f2b, f3w, f3b, blk):
    n = N_NODES
    off = N_ACC // blk
    return pl.pallas_call(
        _head_body,
        grid=(n // blk,),
        in_specs=[
            pl.BlockSpec((blk, 16), lambda i: (i, 0)),
            pl.BlockSpec((blk, 16), lambda i: (i + off, 0)),
            pl.BlockSpec((blk, 8), lambda i: (i, 0)),
            pl.BlockSpec((blk, 8), lambda i: (i + off, 0)),
            pl.BlockSpec((blk, 1), lambda i: (i, 0)),
            pl.BlockSpec((1, b2.shape[1]), lambda i: (0, 0)),
            pl.BlockSpec(f1w.shape, lambda i: (0, 0)),
            pl.BlockSpec((1, f1b.shape[1]), lambda i: (0, 0)),
            pl.BlockSpec(f2w.shape, lambda i: (0, 0)),
            pl.BlockSpec((1, f2b.shape[1]), lambda i: (0, 0)),
            pl.BlockSpec(f3w.shape, lambda i: (0, 0)),
            pl.BlockSpec((1, 1), lambda i: (0, 0)),
        ],
        out_specs=[
            pl.BlockSpec((blk, 1), lambda i: (i, 0)),
            pl.BlockSpec(memory_space=pltpu.SMEM),
            pl.BlockSpec(memory_space=pltpu.SMEM),
        ],
        out_shape=[
            jax.ShapeDtypeStruct((n, 1), jnp.float32),
            jax.ShapeDtypeStruct((1, 1), jnp.float32),
            jax.ShapeDtypeStruct((1, 1), jnp.float32),
        ],
        scratch_shapes=[pltpu.SMEM((1,), jnp.float32),
                        pltpu.SMEM((1,), jnp.float32)],
    )(agg2, agg2, degp, degp, mask, b2, f1w, f1b, f2w, f2b, f3w, f3b)


def _norm_body(logits_ref, m_ref, s_ref, probs_ref):
    logit = logits_ref[...]
    m = m_ref[0, 0]
    s = s_ref[0, 0]
    e = jnp.where(logit == -jnp.inf, 0.0, jnp.exp(logit - m))
    probs_ref[...] = e / s


def _norm(logits, m, s, blk):
    n = logits.shape[0]
    return pl.pallas_call(
        _norm_body,
        grid=(n // blk,),
        in_specs=[
            pl.BlockSpec((blk, 1), lambda i: (i, 0)),
            pl.BlockSpec(memory_space=pltpu.SMEM),
            pl.BlockSpec(memory_space=pltpu.SMEM),
        ],
        out_specs=pl.BlockSpec((blk, 1), lambda i: (i, 0)),
        out_shape=jax.ShapeDtypeStruct((n, 1), jnp.float32),
    )(logits, m, s)


def kernel(real_features, cat_features, edge_index, mask, emb_table,
           W1, b1, W2, b2, fc1_w, fc1_b, fc2_w, fc2_b, fc3_w, fc3_b):
    n = real_features.shape[0]
    r = real_features.shape[1]
    e = edge_index.shape[1]
    blk = N_BLOCK if n % N_BLOCK == 0 else n
    cat = cat_features.astype(jnp.int32)
    src = edge_index[0].astype(jnp.int32)
    dst = edge_index[1].astype(jnp.int32)

    grp = 32 * 2 * CHUNK_G * E_GROUP
    e_pad = ((e + grp - 1) // grp) * grp
    src_p = jnp.concatenate([src, jnp.zeros((e_pad - e,), jnp.int32)])
    dst_p = jnp.concatenate([dst, jnp.full((e_pad - e,), n, jnp.int32)])
    src2 = src_p.reshape(e_pad // E_GROUP, E_GROUP)
    dst2 = dst_p.reshape(e_pad // E_GROUP, E_GROUP)

    zeros16 = jnp.zeros((TILE_ROWS, 16), jnp.float32)
    zeros8 = jnp.zeros((TILE_ROWS, 8), jnp.float32)
    ones8 = jnp.ones((E_GROUP, 8), jnp.float32)

    t1 = emb_table @ W1[r:]
    w1a = W1[:r]
    w1a_s = jnp.stack([w1a[:, :16], w1a[:, 16:]])
    t1_s = jnp.stack([t1[:, :16], t1[:, 16:]])
    w2_s = jnp.stack([W2[:, :16], W2[:, 16:]])

    degp = _sc_deg(dst2, ones8, zeros8)

    q1_flat = _front(real_features, cat, w1a_s, t1_s, blk)
    agg1 = _sc_seg16(q1_flat, src2, dst2, zeros16)

    q2_flat = _mid(agg1, degp, b1[None, :], w2_s, blk)
    agg2 = _sc_seg16(q2_flat, src2, dst2, zeros16)

    logits, m, s = _head(agg2, degp, mask, b2[None, :],
                         fc1_w, fc1_b[None, :], fc2_w, fc2_b[None, :],
                         fc3_w, fc3_b[None, :], blk)
    probs = _norm(logits, m, s, blk)
    return probs[:, 0]


# speedup vs baseline: 1.0195x; 1.0016x over previous
import functools

import jax
import jax.numpy as jnp
from jax import lax
from jax.experimental import pallas as pl
from jax.experimental.pallas import tpu as pltpu
from jax.experimental.pallas import tpu_sc as plsc

N_BLOCK = 2000
NEG_SLOPE = 0.01

N_NODES = 100000
TILE_ROWS = 6500
N_ACC = 16 * TILE_ROWS
E_GROUP = 128
CHUNK_G = 4


def _lrelu(x):
    return jnp.where(x > 0, x, NEG_SLOPE * x)


def _seg16_body(q_hbm, src_hbm, dst_hbm, zeros_hbm, out_hbm,
                sidx0, didx0, rows0, sidx1, didx1, rows1, acc,
                gsem0, ssem0, isem0, gsem1, ssem1, isem1):
    c = lax.axis_index("c")
    s = lax.axis_index("s")
    n_groups = src_hbm.shape[0]
    gpt = n_groups // 16
    n_chunks = gpt // CHUNK_G
    n_pairs = n_chunks // 2

    pltpu.sync_copy(zeros_hbm.at[pl.ds(0, TILE_ROWS)],
                    acc.at[pl.ds(s * TILE_ROWS, TILE_ROWS)])
    plsc.subcore_barrier()

    row_off = c * N_NODES
    bufs = ((sidx0, didx0, rows0, gsem0, ssem0, isem0),
            (sidx1, didx1, rows1, gsem1, ssem1, isem1))

    def load_and_fire(cj, sidx, didx, rows, gsem, isem):
        gb = s * gpt + cj * CHUNK_G
        ic0 = pltpu.async_copy(src_hbm.at[pl.ds(gb, CHUNK_G)], sidx, isem)
        ic1 = pltpu.async_copy(dst_hbm.at[pl.ds(gb, CHUNK_G)], didx, isem)
        ic0.wait()
        ic1.wait()
        for g in range(CHUNK_G):
            for k in range(E_GROUP // 16):
                v = sidx[g, pl.ds(k * 16, 16)]
                sidx[g, pl.ds(k * 16, 16)] = v + row_off
        return [pltpu.async_copy(q_hbm.at[sidx.at[g]],
                                 rows.at[pl.ds(g * E_GROUP, E_GROUP)], gsem)
                for g in range(CHUNK_G)]

    def fire_scatters(didx, rows, ssem):
        return [pltpu.async_copy(rows.at[pl.ds(g * E_GROUP, E_GROUP)],
                                 acc.at[didx.at[g]], ssem, add=True)
                for g in range(CHUNK_G)]

    g0 = load_and_fire(0, bufs[0][0], bufs[0][1], bufs[0][2], bufs[0][3],
                       bufs[0][5])
    for cp in g0:
        cp.wait()

    def pair(j0, carry):
        s0 = fire_scatters(bufs[0][1], bufs[0][2], bufs[0][4])
        g1 = load_and_fire(2 * j0 + 1, bufs[1][0], bufs[1][1], bufs[1][2],
                           bufs[1][3], bufs[1][5])
        for cp in s0:
            cp.wait()

        @pl.when(j0 + 1 < n_pairs)
        def _():
            gn = load_and_fire(2 * j0 + 2, bufs[0][0], bufs[0][1],
                               bufs[0][2], bufs[0][3], bufs[0][5])
            for cp in g1:
                cp.wait()
            s1 = fire_scatters(bufs[1][1], bufs[1][2], bufs[1][4])
            for cp in s1:
                cp.wait()
            for cp in gn:
                cp.wait()

        @pl.when(j0 + 1 >= n_pairs)
        def _():
            for cp in g1:
                cp.wait()
            s1 = fire_scatters(bufs[1][1], bufs[1][2], bufs[1][4])
            for cp in s1:
                cp.wait()

        return carry

    lax.fori_loop(0, n_pairs, pair, 0)
    plsc.subcore_barrier()
    pltpu.sync_copy(acc.at[pl.ds(s * TILE_ROWS, TILE_ROWS)],
                    out_hbm.at[pl.ds(c * N_ACC + s * TILE_ROWS, TILE_ROWS)])


def _sc_seg16(q_flat, src2, dst2, zeros16):
    mesh = plsc.VectorSubcoreMesh(core_axis_name="c", subcore_axis_name="s")
    fn = pl.kernel(
        _seg16_body,
        out_type=jax.ShapeDtypeStruct((2 * N_ACC, 16), jnp.float32),
        mesh=mesh,
        compiler_params=pltpu.CompilerParams(use_tc_tiling_on_sc=False),
        scratch_types=[
            pltpu.VMEM((CHUNK_G, E_GROUP), jnp.int32),
            pltpu.VMEM((CHUNK_G, E_GROUP), jnp.int32),
            pltpu.VMEM((CHUNK_G * E_GROUP, 16), jnp.float32),
            pltpu.VMEM((CHUNK_G, E_GROUP), jnp.int32),
            pltpu.VMEM((CHUNK_G, E_GROUP), jnp.int32),
            pltpu.VMEM((CHUNK_G * E_GROUP, 16), jnp.float32),
            pltpu.VMEM_SHARED((N_ACC, 16), jnp.float32),
            pltpu.SemaphoreType.DMA,
            pltpu.SemaphoreType.DMA,
            pltpu.SemaphoreType.DMA,
            pltpu.SemaphoreType.DMA,
            pltpu.SemaphoreType.DMA,
            pltpu.SemaphoreType.DMA,
        ],
    )
    return fn(q_flat, src2, dst2, zeros16)


def _deg_body(dst_hbm, ones_hbm, zeros_hbm, out_hbm,
              didx, ones_v, acc, ssem):
    c = lax.axis_index("c")
    s = lax.axis_index("s")
    w = c * 16 + s
    n_groups = dst_hbm.shape[0]
    gpt = n_groups // 32
    n_chunks = gpt // CHUNK_G

    pltpu.sync_copy(zeros_hbm.at[pl.ds(0, TILE_ROWS)],
                    acc.at[pl.ds(s * TILE_ROWS, TILE_ROWS)])
    pltpu.sync_copy(ones_hbm, ones_v)
    plsc.subcore_barrier()

    def chunk(j, carry):
        gb = w * gpt + j * CHUNK_G
        pltpu.sync_copy(dst_hbm.at[pl.ds(gb, CHUNK_G)], didx)
        scps = [pltpu.async_copy(ones_v, acc.at[didx.at[g]], ssem, add=True)
                for g in range(CHUNK_G)]
        for cp in scps:
            cp.wait()
        return carry

    lax.fori_loop(0, n_chunks, chunk, 0)
    plsc.subcore_barrier()
    pltpu.sync_copy(acc.at[pl.ds(s * TILE_ROWS, TILE_ROWS)],
                    out_hbm.at[pl.ds(c * N_ACC + s * TILE_ROWS, TILE_ROWS)])


def _sc_deg(dst2, ones8, zeros8):
    mesh = plsc.VectorSubcoreMesh(core_axis_name="c", subcore_axis_name="s")
    fn = pl.kernel(
        _deg_body,
        out_type=jax.ShapeDtypeStruct((2 * N_ACC, 8), jnp.float32),
        mesh=mesh,
        compiler_params=pltpu.CompilerParams(use_tc_tiling_on_sc=False),
        scratch_types=[
            pltpu.VMEM((CHUNK_G, E_GROUP), jnp.int32),
            pltpu.VMEM((E_GROUP, 8), jnp.float32),
            pltpu.VMEM_SHARED((N_ACC, 8), jnp.float32),
            pltpu.SemaphoreType.DMA,
        ],
    )
    return fn(dst2, ones8, zeros8)


def _front_body(real_ref, cat_ref, w1a_ref, t1_ref, q1_ref):
    real = real_ref[...]
    cat = cat_ref[...]
    n_cat = t1_ref.shape[1]
    onehot = (cat == lax.broadcasted_iota(jnp.int32, (1, n_cat), 1)
              ).astype(jnp.float32)
    q1 = jnp.dot(real, w1a_ref[0], preferred_element_type=jnp.float32)
    q1 += jnp.dot(onehot, t1_ref[0], preferred_element_type=jnp.float32)
    q1_ref[...] = q1


def _front(real, cat, w1a_s, t1_s, blk):
    n = real.shape[0]
    nb = n // blk
    return pl.pallas_call(
        _front_body,
        grid=(2, nb),
        in_specs=[
            pl.BlockSpec((blk, real.shape[1]), lambda h, i: (i, 0)),
            pl.BlockSpec((blk, 1), lambda h, i: (i, 0)),
            pl.BlockSpec((1,) + w1a_s.shape[1:], lambda h, i: (h, 0, 0)),
            pl.BlockSpec((1,) + t1_s.shape[1:], lambda h, i: (h, 0, 0)),
        ],
        out_specs=pl.BlockSpec((blk, 16), lambda h, i: (h * nb + i, 0)),
        out_shape=jax.ShapeDtypeStruct((2 * n, 16), jnp.float32),
    )(real, cat, w1a_s, t1_s)


def _mid_body(lo_ref, hi_ref, d0_ref, d1_ref, b1_ref, w2_ref, q2_ref):
    agg = jnp.concatenate([lo_ref[...], hi_ref[...]], axis=1)
    deg = jnp.maximum(d0_ref[...][:, :1] + d1_ref[...][:, :1], 1.0)
    h1 = _lrelu(agg / deg + b1_ref[...])
    q2_ref[...] = jnp.dot(h1, w2_ref[0], preferred_element_type=jnp.float32)


def _mid(agg1, degp, b1, w2_s, blk):
    n = N_NODES
    nb = n // blk
    off = N_ACC // blk
    return pl.pallas_call(
        _mid_body,
        grid=(2, nb),
        in_specs=[
            pl.BlockSpec((blk, 16), lambda h, i: (i, 0)),
            pl.BlockSpec((blk, 16), lambda h, i: (i + off, 0)),
            pl.BlockSpec((blk, 8), lambda h, i: (i, 0)),
            pl.BlockSpec((blk, 8), lambda h, i: (i + off, 0)),
            pl.BlockSpec((1, b1.shape[1]), lambda h, i: (0, 0)),
            pl.BlockSpec((1,) + w2_s.shape[1:], lambda h, i: (h, 0, 0)),
        ],
        out_specs=pl.BlockSpec((blk, 16), lambda h, i: (h * nb + i, 0)),
        out_shape=jax.ShapeDtypeStruct((2 * n, 16), jnp.float32),
    )(agg1, agg1, degp, degp, b1, w2_s)


def _head_body(lo_ref, hi_ref, d0_ref, d1_ref, mask_ref, b2_ref,
               f1w_ref, f1b_ref, f2w_ref, f2b_ref, f3w_ref, f3b_ref,
               logits_ref, m_ref, s_ref, m_sc, s_sc):
    i = pl.program_id(0)

    @pl.when(i == 0)
    def _():
        m_sc[0] = -jnp.inf
        s_sc[0] = 0.0

    agg = jnp.concatenate([lo_ref[...], hi_ref[...]], axis=1)
    deg = jnp.maximum(d0_ref[...][:, :1] + d1_ref[...][:, :1], 1.0)
    embs = agg / deg + b2_ref[...]
    x = _lrelu(jnp.dot(embs, f1w_ref[...], preferred_element_type=jnp.float32)
               + f1b_ref[...])
    x = _lrelu(jnp.dot(x, f2w_ref[...], preferred_element_type=jnp.float32)
               + f2b_ref[...])
    logit = jnp.dot(x, f3w_ref[...], preferred_element_type=jnp.float32) \
        + f3b_ref[...]
    logit = jnp.where(mask_ref[...], logit, -jnp.inf)
    logits_ref[...] = logit

    blk_max = jnp.max(logit)
    m_old = m_sc[0]
    m_new = jnp.maximum(m_old, blk_max)
    scale = jnp.where(m_new == -jnp.inf, 0.0, jnp.exp(m_old - m_new))
    blk_sum = jnp.sum(jnp.where(logit == -jnp.inf, 0.0,
                                jnp.exp(logit - m_new)))
    s_sc[0] = s_sc[0] * scale + blk_sum
    m_sc[0] = m_new

    @pl.when(i == pl.num_programs(0) - 1)
    def _():
        m_ref[0, 0] = m_sc[0]
        s_ref[0, 0] = s_sc[0]


def _head(agg2, degp, mask, b2, f1w, f1b, f2w, f2b, f3w, f3b, blk):
    n = N_NODES
    off = N_ACC // blk
    return pl.pallas_call(
        _head_body,
        grid=(n // blk,),
        in_specs=[
            pl.BlockSpec((blk, 16), lambda i: (i, 0)),
            pl.BlockSpec((blk, 16), lambda i: (i + off, 0)),
            pl.BlockSpec((blk, 8), lambda i: (i, 0)),
            pl.BlockSpec((blk, 8), lambda i: (i + off, 0)),
            pl.BlockSpec((blk, 1), lambda i: (i, 0)),
            pl.BlockSpec((1, b2.shape[1]), lambda i: (0, 0)),
            pl.BlockSpec(f1w.shape, lambda i: (0, 0)),
            pl.BlockSpec((1, f1b.shape[1]), lambda i: (0, 0)),
            pl.BlockSpec(f2w.shape, lambda i: (0, 0)),
            pl.BlockSpec((1, f2b.shape[1]), lambda i: (0, 0)),
            pl.BlockSpec(f3w.shape, lambda i: (0, 0)),
            pl.BlockSpec((1, 1), lambda i: (0, 0)),
        ],
        out_specs=[
            pl.BlockSpec((blk, 1), lambda i: (i, 0)),
            pl.BlockSpec(memory_space=pltpu.SMEM),
            pl.BlockSpec(memory_space=pltpu.SMEM),
        ],
        out_shape=[
            jax.ShapeDtypeStruct((n, 1), jnp.float32),
            jax.ShapeDtypeStruct((1, 1), jnp.float32),
            jax.ShapeDtypeStruct((1, 1), jnp.float32),
        ],
        scratch_shapes=[pltpu.SMEM((1,), jnp.float32),
                        pltpu.SMEM((1,), jnp.float32)],
    )(agg2, agg2, degp, degp, mask, b2, f1w, f1b, f2w, f2b, f3w, f3b)


def _norm_body(logits_ref, m_ref, s_ref, probs_ref):
    logit = logits_ref[...]
    m = m_ref[0, 0]
    s = s_ref[0, 0]
    e = jnp.where(logit == -jnp.inf, 0.0, jnp.exp(logit - m))
    probs_ref[...] = e / s


def _norm(logits, m, s, blk):
    n = logits.shape[0]
    return pl.pallas_call(
        _norm_body,
        grid=(n // blk,),
        in_specs=[
            pl.BlockSpec((blk, 1), lambda i: (i, 0)),
            pl.BlockSpec(memory_space=pltpu.SMEM),
            pl.BlockSpec(memory_space=pltpu.SMEM),
        ],
        out_specs=pl.BlockSpec((blk, 1), lambda i: (i, 0)),
        out_shape=jax.ShapeDtypeStruct((n, 1), jnp.float32),
    )(logits, m, s)


def kernel(real_features, cat_features, edge_index, mask, emb_table,
           W1, b1, W2, b2, fc1_w, fc1_b, fc2_w, fc2_b, fc3_w, fc3_b):
    n = real_features.shape[0]
    r = real_features.shape[1]
    e = edge_index.shape[1]
    blk = N_BLOCK if n % N_BLOCK == 0 else n
    cat = cat_features.astype(jnp.int32)
    src = edge_index[0].astype(jnp.int32)
    dst = edge_index[1].astype(jnp.int32)

    grp = 32 * 2 * CHUNK_G * E_GROUP
    e_pad = ((e + grp - 1) // grp) * grp
    src_p = jnp.concatenate([src, jnp.zeros((e_pad - e,), jnp.int32)])
    dst_p = jnp.concatenate([dst, jnp.full((e_pad - e,), n, jnp.int32)])
    src2 = src_p.reshape(e_pad // E_GROUP, E_GROUP)
    dst2 = dst_p.reshape(e_pad // E_GROUP, E_GROUP)

    zeros16 = jnp.zeros((TILE_ROWS, 16), jnp.float32)
    zeros8 = jnp.zeros((TILE_ROWS, 8), jnp.float32)
    ones8 = jnp.ones((E_GROUP, 8), jnp.float32)

    t1 = emb_table @ W1[r:]
    w1a = W1[:r]
    w1a_s = jnp.stack([w1a[:, :16], w1a[:, 16:]])
    t1_s = jnp.stack([t1[:, :16], t1[:, 16:]])
    w2_s = jnp.stack([W2[:, :16], W2[:, 16:]])

    q1_flat = _front(real_features, cat, w1a_s, t1_s, blk)
    degp = _sc_deg(dst2, ones8, zeros8)
    agg1 = _sc_seg16(q1_flat, src2, dst2, zeros16)

    q2_flat = _mid(agg1, degp, b1[None, :], w2_s, blk)
    agg2 = _sc_seg16(q2_flat, src2, dst2, zeros16)

    logits, m, s = _head(agg2, degp, mask, b2[None, :],
                         fc1_w, fc1_b[None, :], fc2_w, fc2_b[None, :],
                         fc3_w, fc3_b[None, :], blk)
    probs = _norm(logits, m, s, blk)
    return probs[:, 0]
